# Initial kernel scaffold; baseline (speedup 1.0000x reference)
import jax
import jax.numpy as jnp
from jax.experimental import pallas as pl


def kernel(obs, obs_vel, noise, rule_q_W, rule_q_b, ctx_q_W, ctx_q_b, ctx_k_W, ctx_k_b, emb, w_indiv, w_social, w_noise, sp_conv_w, sp_conv_b, sp_res_w, sp_res_b, tm_conv_w, tm_conv_b, tm_res_w, tm_res_b, spi_conv_w, spi_conv_b, spi_res_w, spi_res_b, tmi_conv_w, tmi_conv_b, tmi_res_w, tmi_res_b):
    raise NotImplementedError("write your pallas kernel here")


# speedup vs baseline: 3.0168x; 3.0168x over previous
import jax
import jax.numpy as jnp
import numpy as np
from jax.experimental import pallas as pl

_F32 = jnp.float32


def _body(obs_r, ov_r, ovT_r, nT_r, cqW_r, cqb_r, ckW_r, ckb_r, rqW_r,
          rqb_r, emb_r, Wbig_r, ball_r, M_r, bt_r, wn_r, G_r, Gb_r, Prow_r,
          out_r):
    obs = obs_r[...]
    ov = ov_r[...]
    ovT = ovT_r[...]
    nT = nT_r[...]

    def dgT(a, b):
        return jax.lax.dot_general(a, b, (((1,), (1,)), ((), ())),
                                   preferred_element_type=_F32)

    cqW = cqW_r[...]
    ckW = ckW_r[...]
    cq = cqb_r[...]
    ck = ckb_r[...]
    for c in range(2):
        cq = cq + jnp.dot(obs[c], cqW[c], preferred_element_type=_F32)
        cq = cq + jnp.dot(ov[c], cqW[c + 2], preferred_element_type=_F32)
        ck = ck + jnp.dot(obs[c], ckW[c], preferred_element_type=_F32)
        ck = ck + jnp.dot(ov[c], ckW[c + 2], preferred_element_type=_F32)
    logits = dgT(cq, ck)
    iota20 = jax.lax.broadcasted_iota(jnp.int32, (20, 20), 1)
    a = logits
    masks = []
    for _ in range(3):
        mx = jnp.max(a, axis=-1, keepdims=True)
        cand = jnp.where(a >= mx, iota20, 1000000)
        idx = jnp.min(cand, axis=-1, keepdims=True)
        oh = iota20 == idx
        masks.append(oh.astype(_F32))
        a = jnp.where(oh, -jnp.inf, a)

    rqW = rqW_r[...]
    rq = rqb_r[...]
    for c in range(2):
        rq = rq + jnp.dot(ov[c], rqW[c], preferred_element_type=_F32)
    rlT = dgT(emb_r[...], rq)
    iota8 = jax.lax.broadcasted_iota(jnp.int32, (8, 20), 0)
    mx = jnp.max(rlT, axis=0, keepdims=True)
    cand = jnp.where(rlT >= mx, iota8, 1000000)
    ridx = jnp.min(cand, axis=0, keepdims=True)
    rmaskT = (iota8 == ridx).astype(_F32)

    rows = [ovT[0], ovT[1]]
    for m in masks:
        for c in range(2):
            rows.append(dgT(ovT[c], m))
    combT = jnp.concatenate(rows, axis=0)

    G = G_r[...]
    base = jnp.dot(combT, G, preferred_element_type=_F32)
    nb = jnp.dot(nT, Gb_r[...], preferred_element_type=_F32)
    nbase = jnp.dot(Prow_r[...], nb, preferred_element_type=_F32)

    Wbig = Wbig_r[...]
    ball = ball_r[...]
    M = M_r[...]
    bt = bt_r[...]
    wn = wn_r[...]

    acc = jnp.zeros((24, 400), _F32)
    for r in range(8):
        Xr = base + wn[r:r + 1, 0:1] * nbase
        U = jnp.dot(Wbig[r], Xr, preferred_element_type=_F32) + ball[r]
        Si = jnp.maximum(U[0:16], 0.0) + U[16:32]
        Ss = jnp.maximum(U[32:48], 0.0) + U[48:64]
        A = jnp.concatenate([Si, Ss], axis=0)
        P = jnp.dot(M[r], A, preferred_element_type=_F32) + bt[r]
        mbn = jnp.dot(rmaskT[r:r + 1, :], G, preferred_element_type=_F32)
        acc = acc + mbn * P
    out_r[...] = acc


def kernel(obs, obs_vel, noise, rule_q_W, rule_q_b, ctx_q_W, ctx_q_b,
           ctx_k_W, ctx_k_b, emb, w_indiv, w_social, w_noise,
           sp_conv_w, sp_conv_b, sp_res_w, sp_res_b,
           tm_conv_w, tm_conv_b, tm_res_w, tm_res_b,
           spi_conv_w, spi_conv_b, spi_res_w, spi_res_b,
           tmi_conv_w, tmi_conv_b, tmi_res_w, tmi_res_b):
    obs3 = obs[0]
    ov3 = obs_vel[0]
    ovT = jnp.transpose(ov3, (0, 2, 1))
    nT = noise.reshape(20, 2).T

    cqW = ctx_q_W.reshape(32, 8, 4).transpose(2, 1, 0)
    ckW = ctx_k_W.reshape(32, 8, 4).transpose(2, 1, 0)
    rqW = rule_q_W.reshape(32, 8, 2).transpose(2, 1, 0)
    cqb = ctx_q_b[None, :]
    ckb = ctx_k_b[None, :]
    rqb = rule_q_b[None, :]

    sp_f = sp_conv_w.transpose(0, 1, 3, 2).reshape(8, 2, 24)
    spi_t = spi_conv_w.transpose(0, 1, 3, 2)
    spi_f = jnp.pad(spi_t, ((0, 0), (0, 0), (0, 0), (0, 6))).reshape(8, 2, 24)
    spi_rz = jnp.pad(spi_res_w[..., 0], ((0, 0), (0, 0), (8, 14)))
    sp_rz = jnp.pad(sp_res_w[..., 0], ((0, 0), (0, 0), (8, 8)))
    W1 = jnp.concatenate([spi_f, spi_rz, sp_f, sp_rz], axis=1)
    E = jnp.stack([jnp.eye(8, k=-1, dtype=_F32),
                   jnp.eye(8, dtype=_F32),
                   jnp.eye(8, k=1, dtype=_F32)])
    Wbig = jnp.einsum('rqdc,dtu->rqtcu', W1.reshape(8, 8, 3, 8),
                      E).reshape(8, 64, 64)
    ball = jnp.concatenate([spi_conv_b, spi_res_b, sp_conv_b, sp_res_b],
                           axis=1)
    ball64 = jnp.repeat(ball, 8, axis=1)[..., None]

    def stage2(conv_w, res_w):
        W = conv_w
        Wr = res_w[..., 0]
        M0 = jnp.concatenate([W[..., 1] + Wr, W[..., 2]], axis=2)
        M1 = jnp.concatenate([W[..., 0], W[..., 1] + Wr], axis=2)
        return M0, M1

    Mi0, Mi1 = stage2(tmi_conv_w, tmi_res_w)
    Ms0, Ms1 = stage2(tm_conv_w, tm_res_w)
    wi = w_indiv[:, :, None]
    ws = w_social[:, :, None]
    M0 = jnp.concatenate([wi * Mi0, ws * Ms0], axis=2)
    M1 = jnp.concatenate([wi * Mi1, ws * Ms1], axis=2)
    M = jnp.concatenate([M0, M1], axis=1)
    bt0 = w_indiv * (tmi_conv_b + tmi_res_b) + w_social * (tm_conv_b + tm_res_b)
    bt = jnp.concatenate([bt0, bt0], axis=1)[..., None]

    G = jnp.asarray(np.tile(np.eye(20, dtype=np.float32), (1, 20)))
    Gb = jnp.asarray(np.repeat(np.eye(20, dtype=np.float32), 20, axis=1))
    Prow = jnp.asarray(np.tile(np.eye(2, dtype=np.float32), (4, 1)))
    Prow = jnp.repeat(Prow, 8, axis=0)

    acc = pl.pallas_call(
        _body,
        out_shape=jax.ShapeDtypeStruct((24, 400), _F32),
    )(obs3, ov3, ovT, nT, cqW, cqb, ckW, ckb, rqW, rqb, emb,
      Wbig, ball64, M, bt, w_noise, G, Gb, Prow)
    return jnp.transpose(acc.reshape(2, 12, 20, 20), (2, 3, 1, 0))
